# baseline (speedup 1.0000x reference)
import functools

import jax
import jax.numpy as jnp
from jax import lax
from jax.experimental import pallas as pl
from jax.experimental.pallas import tpu as pltpu
from jax.experimental.pallas import tpu_sc as plsc

_NC = 2
_NS = 16
_NW = _NC * _NS
_CHUNK = 128
_PACK = 4


@functools.cache
def _gather_fn(B, V4):
    b_per_w = B // _NW
    n_chunks = b_per_w // _CHUNK
    mesh = plsc.VectorSubcoreMesh(core_axis_name="c", subcore_axis_name="s")

    @functools.partial(
        pl.kernel,
        out_type=[
            jax.ShapeDtypeStruct((B, 128), jnp.float32),
            jax.ShapeDtypeStruct((B, 128), jnp.float32),
        ],
        mesh=mesh,
        scratch_types=[
            pltpu.VMEM((n_chunks, _CHUNK), jnp.int32),
            pltpu.VMEM((n_chunks, _CHUNK), jnp.int32),
            pltpu.VMEM((b_per_w, 128), jnp.float32),
            pltpu.SemaphoreType.DMA,
        ],
    )
    def gather(uids_hbm, utab_hbm, iids_hbm, itab_hbm, uout_hbm, iout_hbm,
               ids_v, idx_v, rows_v, sem):
        wid = lax.axis_index("s") * _NC + lax.axis_index("c")
        base = wid * b_per_w
        for ids_hbm, tab_hbm, out_hbm in (
                (uids_hbm, utab_hbm, uout_hbm),
                (iids_hbm, itab_hbm, iout_hbm)):
            for j in range(n_chunks):
                pltpu.sync_copy(ids_hbm.at[pl.ds(base + j * _CHUNK, _CHUNK)],
                                ids_v.at[j])
            for j in range(n_chunks):
                for k in range(_CHUNK // 16):
                    sl = pl.ds(k * 16, 16)
                    idx_v[j, sl] = lax.shift_right_logical(ids_v[j, sl], 2)
            copies = []
            for j in range(n_chunks):
                copies.append(pltpu.async_copy(
                    tab_hbm.at[idx_v.at[j]],
                    rows_v.at[pl.ds(j * _CHUNK, _CHUNK)], sem))
            for cp in copies:
                cp.wait()
            pltpu.sync_copy(rows_v, out_hbm.at[pl.ds(base, b_per_w)])

    return gather


def _mlp_body(ur_ref, ir_ref, uid_ref, iid_ref, w1u_ref, w1i_ref, b1_ref,
              w2_ref, b2_ref, wo_ref, bo_ref, out_ref):
    D = 32
    uq = uid_ref[...] & (_PACK - 1)
    iq = iid_ref[...] & (_PACK - 1)
    ue = jnp.zeros(ur_ref.shape[:1] + (D,), jnp.float32)
    ie = jnp.zeros(ur_ref.shape[:1] + (D,), jnp.float32)
    for t in range(_PACK):
        ue = ue + jnp.where(uq == t, ur_ref[:, t * D:(t + 1) * D], 0.0)
        ie = ie + jnp.where(iq == t, ir_ref[:, t * D:(t + 1) * D], 0.0)
    x1 = jnp.dot(ue, w1u_ref[...], preferred_element_type=jnp.float32)
    x2 = jnp.dot(ie, w1i_ref[...], preferred_element_type=jnp.float32)
    h = jnp.maximum(x1 + x2 + b1_ref[...], 0.0)
    h = jnp.maximum(
        jnp.dot(h, w2_ref[...], preferred_element_type=jnp.float32)
        + b2_ref[...], 0.0)
    out_ref[...] = jnp.sum(h * wo_ref[...], axis=1) + bo_ref[0]


@functools.cache
def _mlp_fn(B, D, H1, H2, bb):
    grid = B // bb
    return pl.pallas_call(
        _mlp_body,
        grid=(grid,),
        in_specs=[
            pl.BlockSpec((bb, 128), lambda i: (i, 0)),
            pl.BlockSpec((bb, 128), lambda i: (i, 0)),
            pl.BlockSpec((bb, 1), lambda i: (i, 0)),
            pl.BlockSpec((bb, 1), lambda i: (i, 0)),
            pl.BlockSpec((D, H1), lambda i: (0, 0)),
            pl.BlockSpec((D, H1), lambda i: (0, 0)),
            pl.BlockSpec((1, H1), lambda i: (0, 0)),
            pl.BlockSpec((H1, H2), lambda i: (0, 0)),
            pl.BlockSpec((1, H2), lambda i: (0, 0)),
            pl.BlockSpec((1, H2), lambda i: (0, 0)),
            pl.BlockSpec((1,), lambda i: (0,)),
        ],
        out_specs=pl.BlockSpec((bb,), lambda i: (i,)),
        out_shape=jax.ShapeDtypeStruct((B,), jnp.float32),
    )


def kernel(user_ids, item_ids, user_table, item_table, W1, b1, W2, b2, Wo, bo):
    B = user_ids.shape[0]
    V, D = user_table.shape
    H1 = W1.shape[1]
    H2 = W2.shape[1]

    utab4 = user_table.reshape(V // _PACK, _PACK * D)
    itab4 = item_table.reshape(item_table.shape[0] // _PACK, _PACK * D)
    ur, ir = _gather_fn(B, V // _PACK)(user_ids, utab4, item_ids, itab4)

    out = _mlp_fn(B, D, H1, H2, 512)(
        ur, ir, user_ids.reshape(B, 1), item_ids.reshape(B, 1),
        W1[:D], W1[D:], b1.reshape(1, H1), W2, b2.reshape(1, H2),
        Wo.reshape(1, H2), bo)
    return out


# speedup vs baseline: 3.5487x; 3.5487x over previous
import functools

import jax
import jax.numpy as jnp
from jax import lax
from jax.experimental import pallas as pl
from jax.experimental.pallas import tpu as pltpu
from jax.experimental.pallas import tpu_sc as plsc

_NC = 2
_NS = 16
_NW = _NC * _NS
_CHUNK = 128


_G = 16


@functools.cache
def _gather_fn(B, D, V):
    b_per_w = B // _NW
    n_groups = b_per_w // _G
    mesh = plsc.VectorSubcoreMesh(core_axis_name="c", subcore_axis_name="s")

    @functools.partial(
        pl.kernel,
        out_type=[
            jax.ShapeDtypeStruct((D, B), jnp.float32),
            jax.ShapeDtypeStruct((D, B), jnp.float32),
        ],
        mesh=mesh,
        scratch_types=[
            pltpu.VMEM((b_per_w,), jnp.int32),
            pltpu.VMEM((_G, D, 128), jnp.float32),
            pltpu.VMEM((D, b_per_w), jnp.float32),
            pltpu.SemaphoreType.DMA,
        ],
        compiler_params=pltpu.CompilerParams(needs_layout_passes=False),
    )
    def gather(uids_hbm, utabT_hbm, iids_hbm, itabT_hbm, uoutT_hbm,
               ioutT_hbm, ids_v, slab_v, outT_v, sem):
        wid = lax.axis_index("s") * _NC + lax.axis_index("c")
        base = wid * b_per_w
        for ids_hbm, tabT_hbm, outT_hbm in (
                (uids_hbm, utabT_hbm, uoutT_hbm),
                (iids_hbm, itabT_hbm, ioutT_hbm)):
            pltpu.sync_copy(ids_hbm.at[pl.ds(base, b_per_w)], ids_v)

            def group(g, _):
                gids = ids_v[pl.ds(g * _G, 16)]
                slabs = lax.shift_right_logical(gids, 7) * 128
                for k in range(_G):
                    off = pl.multiple_of(slabs[k], 128)
                    pltpu.async_copy(
                        tabT_hbm.at[:, pl.ds(off, 128)], slab_v.at[k], sem)
                for k in range(_G):
                    pltpu.make_async_copy(
                        tabT_hbm.at[:, pl.ds(0, 128)], slab_v.at[k],
                        sem).wait()
                kvec = lax.iota(jnp.int32, 16)
                lvec = gids & 127
                for c in range(D):
                    cvec = jnp.full((16,), c, jnp.int32)
                    outT_v[c, pl.ds(g * _G, 16)] = plsc.load_gather(
                        slab_v, [kvec, cvec, lvec])
                return 0

            lax.fori_loop(0, n_groups, group, 0)
            pltpu.sync_copy(outT_v, outT_hbm.at[:, pl.ds(base, b_per_w)])

    return gather


def _mlp_body(ueT_ref, ieT_ref, w1u_ref, w1i_ref, b1_ref, w2_ref, b2_ref,
              wo_ref, bo_ref, out_ref):
    contract0 = (((0,), (0,)), ((), ()))
    x1 = lax.dot_general(w1u_ref[...], ueT_ref[...], contract0,
                         preferred_element_type=jnp.float32)
    x2 = lax.dot_general(w1i_ref[...], ieT_ref[...], contract0,
                         preferred_element_type=jnp.float32)
    h = jnp.maximum(x1 + x2 + b1_ref[...], 0.0)
    h = jnp.maximum(
        lax.dot_general(w2_ref[...], h, contract0,
                        preferred_element_type=jnp.float32) + b2_ref[...],
        0.0)
    out_ref[...] = jnp.sum(h * wo_ref[...], axis=0) + bo_ref[0]


@functools.cache
def _mlp_fn(B, D, H1, H2, bb):
    grid = B // bb
    return pl.pallas_call(
        _mlp_body,
        grid=(grid,),
        in_specs=[
            pl.BlockSpec((D, bb), lambda i: (0, i)),
            pl.BlockSpec((D, bb), lambda i: (0, i)),
            pl.BlockSpec((D, H1), lambda i: (0, 0)),
            pl.BlockSpec((D, H1), lambda i: (0, 0)),
            pl.BlockSpec((H1, 1), lambda i: (0, 0)),
            pl.BlockSpec((H1, H2), lambda i: (0, 0)),
            pl.BlockSpec((H2, 1), lambda i: (0, 0)),
            pl.BlockSpec((H2, 1), lambda i: (0, 0)),
            pl.BlockSpec((1,), lambda i: (0,)),
        ],
        out_specs=pl.BlockSpec((bb,), lambda i: (i,)),
        out_shape=jax.ShapeDtypeStruct((B,), jnp.float32),
    )


def kernel(user_ids, item_ids, user_table, item_table, W1, b1, W2, b2, Wo, bo):
    B = user_ids.shape[0]
    V, D = user_table.shape
    H1 = W1.shape[1]
    H2 = W2.shape[1]

    ueT, ieT = _gather_fn(B, D, V)(
        user_ids, user_table.T, item_ids, item_table.T)

    out = _mlp_fn(B, D, H1, H2, 512)(
        ueT, ieT, W1[:D], W1[D:], b1.reshape(H1, 1), W2, b2.reshape(H2, 1),
        Wo, bo)
    return out


# speedup vs baseline: 3.5690x; 1.0057x over previous
import functools

import jax
import jax.numpy as jnp
from jax import lax
from jax.experimental import pallas as pl
from jax.experimental.pallas import tpu as pltpu
from jax.experimental.pallas import tpu_sc as plsc

_NC = 2
_NS = 16
_NW = _NC * _NS
_CHUNK = 128


_G = 8


@functools.cache
def _gather_fn(B, D, V):
    b_per_w = B // _NW
    n_groups = b_per_w // _G
    mesh = plsc.VectorSubcoreMesh(core_axis_name="c", subcore_axis_name="s")

    @functools.partial(
        pl.kernel,
        out_type=[
            jax.ShapeDtypeStruct((D, B), jnp.float32),
            jax.ShapeDtypeStruct((D, B), jnp.float32),
        ],
        mesh=mesh,
        scratch_types=[
            pltpu.VMEM((b_per_w + 16,), jnp.int32),
            pltpu.VMEM((_G, D, 128), jnp.float32),
            pltpu.VMEM((_G, D, 128), jnp.float32),
            pltpu.VMEM((D, b_per_w), jnp.float32),
            pltpu.SemaphoreType.DMA,
            pltpu.SemaphoreType.DMA,
        ],
        compiler_params=pltpu.CompilerParams(needs_layout_passes=False),
    )
    def gather(uids_hbm, utabT_hbm, iids_hbm, itabT_hbm, uoutT_hbm,
               ioutT_hbm, ids_v, slab0_v, slab1_v, outT_v, sem0, sem1):
        wid = lax.axis_index("s") * _NC + lax.axis_index("c")
        base = wid * b_per_w
        kvec = lax.iota(jnp.int32, 16)
        k7 = kvec & 7
        hi = lax.shift_right_logical(kvec, 3)

        for ids_hbm, tabT_hbm, outT_hbm in (
                (uids_hbm, utabT_hbm, uoutT_hbm),
                (iids_hbm, itabT_hbm, ioutT_hbm)):
            pltpu.sync_copy(ids_hbm.at[pl.ds(base, b_per_w)],
                            ids_v.at[pl.ds(0, b_per_w)])

            def fire(g, buf, sem):
                gids = ids_v[pl.ds(g * _G, 16)]
                slabs = lax.shift_right_logical(gids, 7) * 128
                for k in range(_G):
                    off = pl.multiple_of(slabs[k], 128)
                    pltpu.async_copy(
                        tabT_hbm.at[:, pl.ds(off, 128)], buf.at[k], sem)

            def drain(buf, sem):
                for k in range(_G):
                    pltpu.make_async_copy(
                        tabT_hbm.at[:, pl.ds(0, 128)], buf.at[k], sem).wait()

            def extract(g, buf):
                idv = plsc.load_gather(ids_v, [g * _G + k7])
                lvec = idv & 127
                colv = g * _G + k7
                for cp in range(D // 2):
                    cvec = 2 * cp + hi
                    vals = plsc.load_gather(buf, [k7, cvec, lvec])
                    plsc.store_scatter(outT_v, [cvec, colv], vals)

            fire(0, slab0_v, sem0)

            def pair(i, _):
                g0 = 2 * i
                fire(g0 + 1, slab1_v, sem1)
                drain(slab0_v, sem0)
                extract(g0, slab0_v)

                @pl.when(g0 + 2 < n_groups)
                def _():
                    fire(g0 + 2, slab0_v, sem0)

                drain(slab1_v, sem1)
                extract(g0 + 1, slab1_v)
                return 0

            lax.fori_loop(0, n_groups // 2, pair, 0)
            pltpu.sync_copy(outT_v, outT_hbm.at[:, pl.ds(base, b_per_w)])

    return gather


def _mlp_body(ueT_ref, ieT_ref, w1u_ref, w1i_ref, b1_ref, w2_ref, b2_ref,
              wo_ref, bo_ref, out_ref):
    contract0 = (((0,), (0,)), ((), ()))
    x1 = lax.dot_general(w1u_ref[...], ueT_ref[...], contract0,
                         preferred_element_type=jnp.float32)
    x2 = lax.dot_general(w1i_ref[...], ieT_ref[...], contract0,
                         preferred_element_type=jnp.float32)
    h = jnp.maximum(x1 + x2 + b1_ref[...], 0.0)
    h = jnp.maximum(
        lax.dot_general(w2_ref[...], h, contract0,
                        preferred_element_type=jnp.float32) + b2_ref[...],
        0.0)
    out_ref[...] = jnp.sum(h * wo_ref[...], axis=0) + bo_ref[0]


@functools.cache
def _mlp_fn(B, D, H1, H2, bb):
    grid = B // bb
    return pl.pallas_call(
        _mlp_body,
        grid=(grid,),
        in_specs=[
            pl.BlockSpec((D, bb), lambda i: (0, i)),
            pl.BlockSpec((D, bb), lambda i: (0, i)),
            pl.BlockSpec((D, H1), lambda i: (0, 0)),
            pl.BlockSpec((D, H1), lambda i: (0, 0)),
            pl.BlockSpec((H1, 1), lambda i: (0, 0)),
            pl.BlockSpec((H1, H2), lambda i: (0, 0)),
            pl.BlockSpec((H2, 1), lambda i: (0, 0)),
            pl.BlockSpec((H2, 1), lambda i: (0, 0)),
            pl.BlockSpec((1,), lambda i: (0,)),
        ],
        out_specs=pl.BlockSpec((bb,), lambda i: (i,)),
        out_shape=jax.ShapeDtypeStruct((B,), jnp.float32),
    )


def kernel(user_ids, item_ids, user_table, item_table, W1, b1, W2, b2, Wo, bo):
    B = user_ids.shape[0]
    V, D = user_table.shape
    H1 = W1.shape[1]
    H2 = W2.shape[1]

    ueT, ieT = _gather_fn(B, D, V)(
        user_ids, user_table.T, item_ids, item_table.T)

    out = _mlp_fn(B, D, H1, H2, 512)(
        ueT, ieT, W1[:D], W1[D:], b1.reshape(H1, 1), W2, b2.reshape(H2, 1),
        Wo, bo)
    return out
